# baseline (speedup 1.0000x reference)
import functools

import jax
import jax.numpy as jnp
from jax import lax
from jax.experimental import pallas as pl
from jax.experimental.pallas import tpu as pltpu
from jax.experimental.pallas import tpu_sc as plsc

B = 16384
T = 100
D = 32
NC = 2
NS = 16
NW = NC * NS
BW = B // NW
CT = BW // 128
GT = D // 8
MB = BW // 16


def _sc_gather(xt, E):
    mesh = plsc.VectorSubcoreMesh(core_axis_name="c", subcore_axis_name="s")

    @functools.partial(
        pl.kernel,
        mesh=mesh,
        out_type=jax.ShapeDtypeStruct((T, GT, (B // 128) * 1024), jnp.float32),
        compiler_params=pltpu.CompilerParams(
            use_tc_tiling_on_sc=False,
            needs_layout_passes=False,
            skip_device_barrier=True,
        ),
        scratch_types=[
            pltpu.VMEM((T, BW), jnp.int32),
            pltpu.VMEM((BW, D), jnp.float32),
            pltpu.VMEM((BW, D), jnp.float32),
            pltpu.VMEM((GT * CT * 8 * 128,), jnp.float32),
            pltpu.VMEM((GT * CT * 8 * 128,), jnp.float32),
            pltpu.VMEM((D, 16), jnp.int32),
            pltpu.VMEM((D, 16), jnp.int32),
            pltpu.SemaphoreType.DMA,
            pltpu.SemaphoreType.DMA,
            pltpu.SemaphoreType.DMA,
            pltpu.SemaphoreType.DMA,
        ],
    )
    def gather_kernel(xt_hbm, table_hbm, out_hbm, idx_v, rows_a, rows_b,
                      tr_a, tr_b, dtab, wtab, gsem_a, gsem_b, wsem_a, wsem_b):
        wid = lax.axis_index("s") * NC + lax.axis_index("c")
        b0 = wid * BW
        lane = lax.iota(jnp.int32, 16)

        def tab_body(j, carry):
            d_vec = (j + lane) & 31
            dtab[j, :] = d_vec
            wtab[j, :] = ((d_vec >> 3) * 4096 + (d_vec & 7) * 128) + lane
            return carry

        lax.fori_loop(0, D, tab_body, 0)

        pltpu.sync_copy(xt_hbm.at[:, pl.ds(b0, BW)], idx_v)

        def issue_gather(t, rows, gsem):
            pltpu.async_copy(table_hbm.at[idx_v.at[t]], rows, gsem)

        def wait_gather(t, rows, gsem):
            pltpu.make_async_copy(table_hbm.at[idx_v.at[t]], rows, gsem).wait()

        def transpose(rows, tr):
            NI = 8
            def j_body(j, carry):
                d_vecs = [dtab[j + k * (D // NI), :] for k in range(NI)]
                w_vecs = [wtab[j + k * (D // NI), :] for k in range(NI)]
                for m in range(MB):
                    b_vec = lane + m * 16
                    dst_base = (m // 8) * 1024 + (m % 8) * 16
                    vals = [
                        plsc.load_gather(rows, [b_vec, d_vecs[k]])
                        for k in range(NI)
                    ]
                    for k in range(NI):
                        plsc.store_scatter(tr, [w_vecs[k] + dst_base], vals[k])
                return carry

            lax.fori_loop(0, D // NI, j_body, 0)

        def issue_writes(t, tr, wsem):
            for g in range(GT):
                pltpu.async_copy(
                    tr.at[pl.ds(g * CT * 1024, CT * 1024)],
                    out_hbm.at[t, g, pl.ds(wid * CT * 1024, CT * 1024)],
                    wsem,
                )

        def drain_writes(t, tr, wsem):
            for g in range(GT):
                pltpu.make_async_copy(
                    tr.at[pl.ds(g * CT * 1024, CT * 1024)],
                    out_hbm.at[t, g, pl.ds(wid * CT * 1024, CT * 1024)],
                    wsem,
                ).wait()

        def step(t, rows_cur, tr_cur, rows_nxt, gsem_cur, gsem_nxt, wsem_cur):
            @pl.when(t + 1 < T)
            def _():
                issue_gather(t + 1, rows_nxt, gsem_nxt)

            wait_gather(t, rows_cur, gsem_cur)

            @pl.when(t >= 2)
            def _():
                drain_writes(t - 2, tr_cur, wsem_cur)

            transpose(rows_cur, tr_cur)
            issue_writes(t, tr_cur, wsem_cur)

        issue_gather(0, rows_a, gsem_a)

        def pair_body(i, carry):
            t0 = 2 * i
            step(t0, rows_a, tr_a, rows_b, gsem_a, gsem_b, wsem_a)
            step(t0 + 1, rows_b, tr_b, rows_a, gsem_b, gsem_a, wsem_b)
            return carry

        lax.fori_loop(0, T // 2, pair_body, 0)
        drain_writes(T - 2, tr_a, wsem_a)
        drain_writes(T - 1, tr_b, wsem_b)

    return gather_kernel(xt, E)


def kernel(x, E):
    out5 = _sc_gather(x.T, E)
    out5 = out5.reshape(T, GT, B // 128, 8, 128)
    out = jnp.transpose(out5, (2, 4, 0, 1, 3)).reshape(B, T, D)
    return out


# speedup vs baseline: 1.0014x; 1.0014x over previous
import functools

import jax
import jax.numpy as jnp
from jax import lax
from jax.experimental import pallas as pl
from jax.experimental.pallas import tpu as pltpu
from jax.experimental.pallas import tpu_sc as plsc

B = 16384
T = 100
D = 32
NC = 2
NS = 16
NW = NC * NS
BW = B // NW
CT = BW // 128
GT = D // 8
MB = BW // 16


def _sc_gather(xt, E):
    mesh = plsc.VectorSubcoreMesh(core_axis_name="c", subcore_axis_name="s")

    @functools.partial(
        pl.kernel,
        mesh=mesh,
        out_type=jax.ShapeDtypeStruct((T, GT, (B // 128) * 1024), jnp.float32),
        compiler_params=pltpu.CompilerParams(
            use_tc_tiling_on_sc=False, needs_layout_passes=False
        ),
        scratch_types=[
            pltpu.VMEM((T, BW), jnp.int32),
            pltpu.VMEM((BW, D), jnp.float32),
            pltpu.VMEM((BW, D), jnp.float32),
            pltpu.VMEM((GT * CT * 8 * 128,), jnp.float32),
            pltpu.VMEM((GT * CT * 8 * 128,), jnp.float32),
            pltpu.VMEM((D, 16), jnp.int32),
            pltpu.VMEM((D, 16), jnp.int32),
            pltpu.SemaphoreType.DMA,
            pltpu.SemaphoreType.DMA,
            pltpu.SemaphoreType.DMA,
            pltpu.SemaphoreType.DMA,
        ],
    )
    def gather_kernel(xt_hbm, table_hbm, out_hbm, idx_v, rows_a, rows_b,
                      tr_a, tr_b, dtab, wtab, gsem_a, gsem_b, wsem_a, wsem_b):
        wid = lax.axis_index("s") * NC + lax.axis_index("c")
        b0 = wid * BW
        lane = lax.iota(jnp.int32, 16)

        def tab_body(j, carry):
            d_vec = (j + lane) & 31
            dtab[j, :] = d_vec
            wtab[j, :] = ((d_vec >> 3) * 4096 + (d_vec & 7) * 128) + lane
            return carry

        lax.fori_loop(0, D, tab_body, 0)

        pltpu.sync_copy(xt_hbm.at[:, pl.ds(b0, BW)], idx_v)

        def issue_gather(t, rows, gsem):
            pltpu.async_copy(table_hbm.at[idx_v.at[t]], rows, gsem)

        def wait_gather(t, rows, gsem):
            pltpu.make_async_copy(table_hbm.at[idx_v.at[t]], rows, gsem).wait()

        def transpose(rows, tr):
            NI = 8
            def j_body(j, carry):
                d_vecs = [dtab[j + k * (D // NI), :] for k in range(NI)]
                w_vecs = [wtab[j + k * (D // NI), :] for k in range(NI)]
                for m in range(MB):
                    b_vec = lane + m * 16
                    dst_base = (m // 8) * 1024 + (m % 8) * 16
                    vals = [
                        plsc.load_gather(rows, [b_vec, d_vecs[k]])
                        for k in range(NI)
                    ]
                    for k in range(NI):
                        plsc.store_scatter(tr, [w_vecs[k] + dst_base], vals[k])
                return carry

            lax.fori_loop(0, D // NI, j_body, 0)

        def issue_writes(t, tr, wsem):
            for g in range(GT):
                pltpu.async_copy(
                    tr.at[pl.ds(g * CT * 1024, CT * 1024)],
                    out_hbm.at[t, g, pl.ds(wid * CT * 1024, CT * 1024)],
                    wsem,
                )

        def drain_writes(t, tr, wsem):
            for g in range(GT):
                pltpu.make_async_copy(
                    tr.at[pl.ds(g * CT * 1024, CT * 1024)],
                    out_hbm.at[t, g, pl.ds(wid * CT * 1024, CT * 1024)],
                    wsem,
                ).wait()

        def step(t, rows_cur, tr_cur, rows_nxt, gsem_cur, gsem_nxt, wsem_cur):
            @pl.when(t + 1 < T)
            def _():
                issue_gather(t + 1, rows_nxt, gsem_nxt)

            wait_gather(t, rows_cur, gsem_cur)

            @pl.when(t >= 2)
            def _():
                drain_writes(t - 2, tr_cur, wsem_cur)

            transpose(rows_cur, tr_cur)
            issue_writes(t, tr_cur, wsem_cur)

        issue_gather(0, rows_a, gsem_a)

        def pair_body(i, carry):
            t0 = 2 * i
            step(t0, rows_a, tr_a, rows_b, gsem_a, gsem_b, wsem_a)
            step(t0 + 1, rows_b, tr_b, rows_a, gsem_b, gsem_a, wsem_b)
            return carry

        lax.fori_loop(0, T // 2, pair_body, 0)
        drain_writes(T - 2, tr_a, wsem_a)
        drain_writes(T - 1, tr_b, wsem_b)

    return gather_kernel(xt, E)


def kernel(x, E):
    out5 = _sc_gather(x.T, E)
    out5 = out5.reshape(T, GT, B // 128, 8, 128)
    out = jnp.transpose(out5, (2, 4, 0, 1, 3)).reshape(B, T, D)
    return out
